# Initial kernel scaffold; baseline (speedup 1.0000x reference)
import jax
import jax.numpy as jnp
from jax.experimental import pallas as pl


def kernel(msg, emb1, emb2, emb3, W, b):
    raise NotImplementedError("write your pallas kernel here")


# speedup vs baseline: 4.9702x; 4.9702x over previous
import functools

import jax
import jax.numpy as jnp
from jax import lax
from jax.experimental import pallas as pl
from jax.experimental.pallas import tpu as pltpu
from jax.experimental.pallas import tpu_sc as plsc

_VOCAB = 100
_DIM = 64
_ROLES = 8
_BS = 4096
_MAX_LEN = 200

_NGRAMS = 3 * _MAX_LEN - 3
_C12_BI_OFF = 104
_C12_ZERO = 10104
_C12_ROWS = 10112
_IDX_W = 600
_E3_BLOCK = 20000
_NW = 32
_ROWS_PER_W = _BS // _NW

_CHUNKS = (
    (0, 128, 0), (128, 128, 0), (256, 128, 0), (384, 16, 0),
    (400, 104, 1), (504, 94, 1),
)


def _proj_body(e_ref, w_ref, o_ref):
    o_ref[...] = jnp.dot(e_ref[...], w_ref[...],
                         preferred_element_type=jnp.float32)


def _c12_body(e1_ref, e2_ref, w_ref, o_ref):
    w = w_ref[...]
    p1 = jnp.dot(e1_ref[...], w, preferred_element_type=jnp.float32)
    o_ref[0:104, :] = jnp.concatenate(
        [p1, jnp.zeros((4, 16), jnp.float32)], axis=0)
    o_ref[104:10104, :] = jnp.dot(e2_ref[...], w,
                                  preferred_element_type=jnp.float32)
    o_ref[10104:10112, :] = jnp.zeros((8, 16), jnp.float32)


def _idx_body(m_ref, o_ref):
    m = m_ref[...]
    rows = m.shape[0]
    bi = _C12_BI_OFF + m[:, :199] + 100 * m[:, 1:200]
    tri = m[:, :198] + 100 * m[:, 1:199] + 10000 * m[:, 2:200]
    pad1 = jnp.full((rows, 1), _C12_ZERO, jnp.int32)
    pad2 = jnp.full((rows, 2), _C12_ZERO, jnp.int32)
    o_ref[...] = jnp.concatenate([m, bi, pad1, tri, pad2], axis=1)


def _sc_body(c12_hbm, c3_hbm, idx_hbm, b_hbm, out_hbm,
             idx_v, rows_v, outblk_v, b_v, sem):
    pltpu.sync_copy(b_hbm, b_v)
    wid = lax.axis_index("s") * 2 + lax.axis_index("c")
    base = wid * _ROWS_PER_W

    @pl.loop(0, _ROWS_PER_W)
    def _(i):
        pltpu.sync_copy(idx_hbm.at[base + i], idx_v)
        tabs = (c12_hbm, c3_hbm)
        copies = [
            pltpu.async_copy(tabs[t].at[idx_v.at[pl.ds(off, n)]],
                             rows_v.at[pl.ds(off, n)], sem)
            for (off, n, t) in _CHUNKS
        ]
        for c in copies:
            c.wait()

        def add_row(j, acc):
            return acc + rows_v[j, :]

        acc = lax.fori_loop(0, _NGRAMS + 1, add_row,
                            jnp.zeros((16,), jnp.float32))
        outblk_v[i, :] = acc * jnp.float32(1.0 / _NGRAMS) + b_v[...]

    pltpu.sync_copy(outblk_v, out_hbm.at[pl.ds(base, _ROWS_PER_W)])


def kernel(msg, emb1, emb2, emb3, W, b):
    w16 = jnp.pad(W, ((0, 0), (0, 16 - _ROLES)))
    b16 = jnp.pad(b, (0, 16 - _ROLES))

    c3 = pl.pallas_call(
        _proj_body,
        grid=(emb3.shape[0] // _E3_BLOCK,),
        in_specs=[
            pl.BlockSpec((_E3_BLOCK, _DIM), lambda i: (i, 0)),
            pl.BlockSpec((_DIM, 16), lambda i: (0, 0)),
        ],
        out_specs=pl.BlockSpec((_E3_BLOCK, 16), lambda i: (i, 0)),
        out_shape=jax.ShapeDtypeStruct((emb3.shape[0], 16), jnp.float32),
    )(emb3, w16)

    c12 = pl.pallas_call(
        _c12_body,
        out_shape=jax.ShapeDtypeStruct((_C12_ROWS, 16), jnp.float32),
    )(emb1, emb2, w16)

    idx = pl.pallas_call(
        _idx_body,
        grid=(8,),
        in_specs=[pl.BlockSpec((_BS // 8, _MAX_LEN), lambda i: (i, 0))],
        out_specs=pl.BlockSpec((_BS // 8, _IDX_W), lambda i: (i, 0)),
        out_shape=jax.ShapeDtypeStruct((_BS, _IDX_W), jnp.int32),
    )(msg)

    mesh = plsc.VectorSubcoreMesh(core_axis_name="c", subcore_axis_name="s")
    sc = pl.kernel(
        _sc_body,
        mesh=mesh,
        compiler_params=pltpu.CompilerParams(use_tc_tiling_on_sc=False),
        out_type=jax.ShapeDtypeStruct((_BS, 16), jnp.float32),
        scratch_types=[
            pltpu.VMEM((_IDX_W,), jnp.int32),
            pltpu.VMEM((_IDX_W, 16), jnp.float32),
            pltpu.VMEM((_ROWS_PER_W, 16), jnp.float32),
            pltpu.VMEM((16,), jnp.float32),
            pltpu.SemaphoreType.DMA,
        ],
    )
    out16 = sc(c12, c3, idx, b16)
    return out16[:, :_ROLES]


# speedup vs baseline: 6.8458x; 1.3774x over previous
import functools

import jax
import jax.numpy as jnp
from jax import lax
from jax.experimental import pallas as pl
from jax.experimental.pallas import tpu as pltpu
from jax.experimental.pallas import tpu_sc as plsc

_VOCAB = 100
_DIM = 64
_ROLES = 8
_BS = 4096
_MAX_LEN = 200

_NGRAMS = 3 * _MAX_LEN - 3
_C12_BI_OFF = 104
_C12_ZERO = 10104
_C12_ROWS = 10112
_IDX_W = 640
_E3_BLOCK = 5000
_NW = 32
_ROWS_PER_W = _BS // _NW

_CHUNKS = (
    (0, 128, 0), (128, 128, 0), (256, 128, 0), (384, 16, 0),
    (400, 104, 1), (504, 94, 1),
)


def _proj_body(e_ref, w_ref, o_ref):
    o_ref[...] = jnp.dot(e_ref[...], w_ref[...],
                         preferred_element_type=jnp.float32)


def _c12_body(e1_ref, e2_ref, w_ref, o_ref):
    w = w_ref[...]
    p1 = jnp.dot(e1_ref[...], w, preferred_element_type=jnp.float32)
    o_ref[0:104, :] = jnp.concatenate(
        [p1, jnp.zeros((4, 16), jnp.float32)], axis=0)
    o_ref[104:10104, :] = jnp.dot(e2_ref[...], w,
                                  preferred_element_type=jnp.float32)
    o_ref[10104:10112, :] = jnp.zeros((8, 16), jnp.float32)


def _idx_body(m_ref, o_ref):
    m = m_ref[...]
    rows = m.shape[0]
    bi = _C12_BI_OFF + m[:, :199] + 100 * m[:, 1:200]
    tri = m[:, :198] + 100 * m[:, 1:199] + 10000 * m[:, 2:200]
    pad1 = jnp.full((rows, 1), _C12_ZERO, jnp.int32)
    pad2 = jnp.full((rows, _IDX_W - 598), _C12_ZERO, jnp.int32)
    o_ref[...] = jnp.concatenate([m, bi, pad1, tri, pad2], axis=1)


def _sc_body(c12_hbm, c3_hbm, idx_hbm, b_hbm, out_hbm,
             idx_v, rows_v, outblk_v, b_v, sem):
    pltpu.sync_copy(b_hbm, b_v)
    rows_v[598, :] = jnp.zeros((16,), jnp.float32)
    rows_v[599, :] = jnp.zeros((16,), jnp.float32)
    wid = lax.axis_index("s") * 2 + lax.axis_index("c")
    base = wid * _ROWS_PER_W

    @pl.loop(0, _ROWS_PER_W)
    def _(i):
        pltpu.sync_copy(idx_hbm.at[base + i], idx_v)
        tabs = (c12_hbm, c3_hbm)
        copies = [
            pltpu.async_copy(tabs[t].at[idx_v.at[pl.ds(off, n)]],
                             rows_v.at[pl.ds(off, n)], sem)
            for (off, n, t) in _CHUNKS
        ]
        for c in copies:
            c.wait()

        def add4(j, accs):
            a0, a1, a2, a3 = accs
            return (a0 + rows_v[4 * j, :], a1 + rows_v[4 * j + 1, :],
                    a2 + rows_v[4 * j + 2, :], a3 + rows_v[4 * j + 3, :])

        zero = jnp.zeros((16,), jnp.float32)
        a0, a1, a2, a3 = lax.fori_loop(0, 150, add4, (zero, zero, zero, zero))
        acc = (a0 + a1) + (a2 + a3)
        outblk_v[i, :] = acc * jnp.float32(1.0 / _NGRAMS) + b_v[...]

    pltpu.sync_copy(outblk_v, out_hbm.at[pl.ds(base, _ROWS_PER_W)])


def kernel(msg, emb1, emb2, emb3, W, b):
    w16 = jnp.pad(W, ((0, 0), (0, 16 - _ROLES)))
    b16 = jnp.pad(b, (0, 16 - _ROLES))
    w8 = jnp.zeros((512, 128), jnp.float32)
    for k in range(8):
        w8 = w8.at[64 * k:64 * (k + 1), 16 * k:16 * (k + 1)].set(w16)
    e3v = emb3.reshape(emb3.shape[0] // 8, 512)

    c3p = pl.pallas_call(
        _proj_body,
        grid=(e3v.shape[0] // _E3_BLOCK,),
        in_specs=[
            pl.BlockSpec((_E3_BLOCK, 512), lambda i: (i, 0)),
            pl.BlockSpec((512, 128), lambda i: (0, 0)),
        ],
        out_specs=pl.BlockSpec((_E3_BLOCK, 128), lambda i: (i, 0)),
        out_shape=jax.ShapeDtypeStruct((e3v.shape[0], 128), jnp.float32),
    )(e3v, w8)
    c3 = c3p.reshape(emb3.shape[0], 16)

    c12 = pl.pallas_call(
        _c12_body,
        out_shape=jax.ShapeDtypeStruct((_C12_ROWS, 16), jnp.float32),
    )(emb1, emb2, w16)

    idx = pl.pallas_call(
        _idx_body,
        grid=(8,),
        in_specs=[pl.BlockSpec((_BS // 8, _MAX_LEN), lambda i: (i, 0))],
        out_specs=pl.BlockSpec((_BS // 8, _IDX_W), lambda i: (i, 0)),
        out_shape=jax.ShapeDtypeStruct((_BS, _IDX_W), jnp.int32),
    )(msg)

    mesh = plsc.VectorSubcoreMesh(core_axis_name="c", subcore_axis_name="s")
    sc = pl.kernel(
        _sc_body,
        mesh=mesh,
        compiler_params=pltpu.CompilerParams(use_tc_tiling_on_sc=False),
        out_type=jax.ShapeDtypeStruct((_BS, 16), jnp.float32),
        scratch_types=[
            pltpu.VMEM((_IDX_W,), jnp.int32),
            pltpu.VMEM((600, 16), jnp.float32),
            pltpu.VMEM((_ROWS_PER_W, 16), jnp.float32),
            pltpu.VMEM((16,), jnp.float32),
            pltpu.SemaphoreType.DMA,
        ],
    )
    out16 = sc(c12, c3, idx, b16)
    return out16[:, :_ROLES]


# speedup vs baseline: 6.8679x; 1.0032x over previous
import functools

import jax
import jax.numpy as jnp
from jax import lax
from jax.experimental import pallas as pl
from jax.experimental.pallas import tpu as pltpu
from jax.experimental.pallas import tpu_sc as plsc

_VOCAB = 100
_DIM = 64
_ROLES = 8
_BS = 4096
_MAX_LEN = 200

_NGRAMS = 3 * _MAX_LEN - 3
_C12_BI_OFF = 104
_C12_ZERO = 10104
_C12_ROWS = 10112
_IDX_W = 640
_NW = 32
_ROWS_PER_W = _BS // _NW

_CHUNKS16 = ((0, 128), (128, 128), (256, 128), (384, 16))
_CHUNKS64 = ((400, 0, 104), (504, 104, 94))


def _c12_body(e1_ref, e2_ref, w_ref, o_ref):
    w = w_ref[...]
    p1 = jnp.dot(e1_ref[...], w, preferred_element_type=jnp.float32)
    o_ref[0:104, :] = jnp.concatenate(
        [p1, jnp.zeros((4, 16), jnp.float32)], axis=0)
    o_ref[104:10104, :] = jnp.dot(e2_ref[...], w,
                                  preferred_element_type=jnp.float32)
    o_ref[10104:10112, :] = jnp.zeros((8, 16), jnp.float32)


def _idx_body(m_ref, o_ref):
    m = m_ref[...]
    rows = m.shape[0]
    bi = _C12_BI_OFF + m[:, :199] + 100 * m[:, 1:200]
    tri = m[:, :198] + 100 * m[:, 1:199] + 10000 * m[:, 2:200]
    pad1 = jnp.full((rows, 1), _C12_ZERO, jnp.int32)
    pad2 = jnp.full((rows, _IDX_W - 598), _C12_ZERO, jnp.int32)
    o_ref[...] = jnp.concatenate([m, bi, pad1, tri, pad2], axis=1)


def _final_body(s16_ref, s64_ref, w_ref, b_ref, o_ref):
    y = s16_ref[...] + jnp.dot(s64_ref[...], w_ref[...],
                               preferred_element_type=jnp.float32)
    o_ref[...] = y * jnp.float32(1.0 / _NGRAMS) + b_ref[...]


def _sc_body(c12_hbm, e3_hbm, idx_hbm, o16_hbm, o64_hbm,
             idx_v, r16_v, r64_v, blk16_v, blk64_v, sem):
    wid = lax.axis_index("s") * 2 + lax.axis_index("c")
    base = wid * _ROWS_PER_W

    @pl.loop(0, _ROWS_PER_W)
    def _(i):
        pltpu.sync_copy(idx_hbm.at[base + i], idx_v)
        copies = [
            pltpu.async_copy(c12_hbm.at[idx_v.at[pl.ds(off, n)]],
                             r16_v.at[pl.ds(off, n)], sem)
            for (off, n) in _CHUNKS16
        ] + [
            pltpu.async_copy(e3_hbm.at[idx_v.at[pl.ds(off, n)]],
                             r64_v.at[pl.ds(doff, n)], sem)
            for (off, doff, n) in _CHUNKS64
        ]
        for c in copies:
            c.wait()

        def add16(j, accs):
            a0, a1, a2, a3 = accs
            return (a0 + r16_v[4 * j, :], a1 + r16_v[4 * j + 1, :],
                    a2 + r16_v[4 * j + 2, :], a3 + r16_v[4 * j + 3, :])

        zero = jnp.zeros((16,), jnp.float32)
        a0, a1, a2, a3 = lax.fori_loop(0, 100, add16,
                                       (zero, zero, zero, zero))
        blk16_v[i, :] = (a0 + a1) + (a2 + a3)

        def add64(j, accs):
            b0, b1, b2, b3 = accs
            r0 = 2 * j
            r1 = 2 * j + 1
            return (b0 + r64_v[r0, pl.ds(0, 16)] + r64_v[r1, pl.ds(0, 16)],
                    b1 + r64_v[r0, pl.ds(16, 16)] + r64_v[r1, pl.ds(16, 16)],
                    b2 + r64_v[r0, pl.ds(32, 16)] + r64_v[r1, pl.ds(32, 16)],
                    b3 + r64_v[r0, pl.ds(48, 16)] + r64_v[r1, pl.ds(48, 16)])

        b0, b1, b2, b3 = lax.fori_loop(0, 99, add64, (zero, zero, zero, zero))
        blk64_v[i, pl.ds(0, 16)] = b0
        blk64_v[i, pl.ds(16, 16)] = b1
        blk64_v[i, pl.ds(32, 16)] = b2
        blk64_v[i, pl.ds(48, 16)] = b3

    pltpu.sync_copy(blk16_v, o16_hbm.at[pl.ds(base, _ROWS_PER_W)])
    pltpu.sync_copy(blk64_v, o64_hbm.at[pl.ds(base, _ROWS_PER_W)])


def kernel(msg, emb1, emb2, emb3, W, b):
    w16 = jnp.pad(W, ((0, 0), (0, 16 - _ROLES)))
    b16 = jnp.pad(b, (0, 16 - _ROLES)).reshape(1, 16)

    c12 = pl.pallas_call(
        _c12_body,
        out_shape=jax.ShapeDtypeStruct((_C12_ROWS, 16), jnp.float32),
    )(emb1, emb2, w16)

    idx = pl.pallas_call(
        _idx_body,
        grid=(8,),
        in_specs=[pl.BlockSpec((_BS // 8, _MAX_LEN), lambda i: (i, 0))],
        out_specs=pl.BlockSpec((_BS // 8, _IDX_W), lambda i: (i, 0)),
        out_shape=jax.ShapeDtypeStruct((_BS, _IDX_W), jnp.int32),
    )(msg)

    mesh = plsc.VectorSubcoreMesh(core_axis_name="c", subcore_axis_name="s")
    sc = pl.kernel(
        _sc_body,
        mesh=mesh,
        compiler_params=pltpu.CompilerParams(use_tc_tiling_on_sc=False),
        out_type=[
            jax.ShapeDtypeStruct((_BS, 16), jnp.float32),
            jax.ShapeDtypeStruct((_BS, _DIM), jnp.float32),
        ],
        scratch_types=[
            pltpu.VMEM((_IDX_W,), jnp.int32),
            pltpu.VMEM((400, 16), jnp.float32),
            pltpu.VMEM((200, _DIM), jnp.float32),
            pltpu.VMEM((_ROWS_PER_W, 16), jnp.float32),
            pltpu.VMEM((_ROWS_PER_W, _DIM), jnp.float32),
            pltpu.SemaphoreType.DMA,
        ],
    )
    s16, s64 = sc(c12, emb3, idx)

    out16 = pl.pallas_call(
        _final_body,
        grid=(4,),
        in_specs=[
            pl.BlockSpec((_BS // 4, 16), lambda i: (i, 0)),
            pl.BlockSpec((_BS // 4, _DIM), lambda i: (i, 0)),
            pl.BlockSpec((_DIM, 16), lambda i: (0, 0)),
            pl.BlockSpec((1, 16), lambda i: (0, 0)),
        ],
        out_specs=pl.BlockSpec((_BS // 4, 16), lambda i: (i, 0)),
        out_shape=jax.ShapeDtypeStruct((_BS, 16), jnp.float32),
    )(s16, s64, w16, b16)
    return out16[:, :_ROLES]


# speedup vs baseline: 7.7778x; 1.1325x over previous
import functools

import jax
import jax.numpy as jnp
from jax import lax
from jax.experimental import pallas as pl
from jax.experimental.pallas import tpu as pltpu
from jax.experimental.pallas import tpu_sc as plsc

_VOCAB = 100
_DIM = 64
_ROLES = 8
_BS = 4096
_MAX_LEN = 200

_NGRAMS = 3 * _MAX_LEN - 3
_C12_BI_OFF = 104
_C12_ZERO = 10104
_C12_ROWS = 10112
_IDX_W = 640
_E3_BLOCK = 16384
_NW = 32
_ROWS_PER_W = _BS // _NW

_CHUNKS = (
    (0, 128, 0), (128, 128, 0), (256, 128, 0), (384, 16, 0),
    (400, 104, 1), (504, 94, 1),
)


def _proj_body(et_ref, w_ref, o_ref):
    o_ref[...] = lax.dot_general(
        et_ref[...], w_ref[...],
        dimension_numbers=(((0,), (0,)), ((), ())),
        preferred_element_type=jnp.float32)


def _c12_body(e1_ref, e2_ref, w_ref, o_ref):
    w = w_ref[...]
    p1 = jnp.dot(e1_ref[...], w, preferred_element_type=jnp.float32)
    o_ref[0:104, :] = jnp.concatenate(
        [p1, jnp.zeros((4, 16), jnp.float32)], axis=0)
    o_ref[104:10104, :] = jnp.dot(e2_ref[...], w,
                                  preferred_element_type=jnp.float32)
    o_ref[10104:10112, :] = jnp.zeros((8, 16), jnp.float32)


def _idx_body(m_ref, o_ref):
    m = m_ref[...]
    rows = m.shape[0]
    bi = _C12_BI_OFF + m[:, :199] + 100 * m[:, 1:200]
    tri = m[:, :198] + 100 * m[:, 1:199] + 10000 * m[:, 2:200]
    pad1 = jnp.full((rows, 1), _C12_ZERO, jnp.int32)
    pad2 = jnp.full((rows, _IDX_W - 598), _C12_ZERO, jnp.int32)
    o_ref[...] = jnp.concatenate([m, bi, pad1, tri, pad2], axis=1)


def _sc_body(c12_hbm, c3_hbm, idx_hbm, b_hbm, out_hbm,
             idx_v, rows_v, outblk_v, b_v, sem):
    pltpu.sync_copy(b_hbm, b_v)
    rows_v[598, :] = jnp.zeros((16,), jnp.float32)
    rows_v[599, :] = jnp.zeros((16,), jnp.float32)
    wid = lax.axis_index("s") * 2 + lax.axis_index("c")
    base = wid * _ROWS_PER_W

    @pl.loop(0, _ROWS_PER_W)
    def _(i):
        pltpu.sync_copy(idx_hbm.at[base + i], idx_v)
        tabs = (c12_hbm, c3_hbm)
        copies = [
            pltpu.async_copy(tabs[t].at[idx_v.at[pl.ds(off, n)]],
                             rows_v.at[pl.ds(off, n)], sem)
            for (off, n, t) in _CHUNKS
        ]
        for c in copies:
            c.wait()

        def add4(j, accs):
            a0, a1, a2, a3 = accs
            return (a0 + rows_v[4 * j, :], a1 + rows_v[4 * j + 1, :],
                    a2 + rows_v[4 * j + 2, :], a3 + rows_v[4 * j + 3, :])

        zero = jnp.zeros((16,), jnp.float32)
        a0, a1, a2, a3 = lax.fori_loop(0, 150, add4, (zero, zero, zero, zero))
        acc = (a0 + a1) + (a2 + a3)
        outblk_v[i, :] = acc * jnp.float32(1.0 / _NGRAMS) + b_v[...]

    pltpu.sync_copy(outblk_v, out_hbm.at[pl.ds(base, _ROWS_PER_W)])


def kernel(msg, emb1, emb2, emb3, W, b):
    w16 = jnp.pad(W, ((0, 0), (0, 16 - _ROLES)))
    b16 = jnp.pad(b, (0, 16 - _ROLES))
    e3t = emb3.T

    n3 = emb3.shape[0]
    c3 = pl.pallas_call(
        _proj_body,
        grid=((n3 + _E3_BLOCK - 1) // _E3_BLOCK,),
        in_specs=[
            pl.BlockSpec((_DIM, _E3_BLOCK), lambda i: (0, i)),
            pl.BlockSpec((_DIM, 16), lambda i: (0, 0)),
        ],
        out_specs=pl.BlockSpec((_E3_BLOCK, 16), lambda i: (i, 0)),
        out_shape=jax.ShapeDtypeStruct((n3, 16), jnp.float32),
    )(e3t, w16)

    c12 = pl.pallas_call(
        _c12_body,
        out_shape=jax.ShapeDtypeStruct((_C12_ROWS, 16), jnp.float32),
    )(emb1, emb2, w16)

    idx = pl.pallas_call(
        _idx_body,
        grid=(8,),
        in_specs=[pl.BlockSpec((_BS // 8, _MAX_LEN), lambda i: (i, 0))],
        out_specs=pl.BlockSpec((_BS // 8, _IDX_W), lambda i: (i, 0)),
        out_shape=jax.ShapeDtypeStruct((_BS, _IDX_W), jnp.int32),
    )(msg)

    mesh = plsc.VectorSubcoreMesh(core_axis_name="c", subcore_axis_name="s")
    sc = pl.kernel(
        _sc_body,
        mesh=mesh,
        compiler_params=pltpu.CompilerParams(use_tc_tiling_on_sc=False),
        out_type=jax.ShapeDtypeStruct((_BS, 16), jnp.float32),
        scratch_types=[
            pltpu.VMEM((_IDX_W,), jnp.int32),
            pltpu.VMEM((600, 16), jnp.float32),
            pltpu.VMEM((_ROWS_PER_W, 16), jnp.float32),
            pltpu.VMEM((16,), jnp.float32),
            pltpu.SemaphoreType.DMA,
        ],
    )
    out16 = sc(c12, c3, idx, b16)
    return out16[:, :_ROLES]


# speedup vs baseline: 10.2250x; 1.3146x over previous
import functools

import jax
import jax.numpy as jnp
from jax import lax
from jax.experimental import pallas as pl
from jax.experimental.pallas import tpu as pltpu
from jax.experimental.pallas import tpu_sc as plsc

_VOCAB = 100
_DIM = 64
_ROLES = 8
_BS = 4096
_MAX_LEN = 200

_NGRAMS = 3 * _MAX_LEN - 3
_C12_BI_OFF = 104
_C12_ZERO = 10104
_C12_ROWS = 10112
_IDX_W = 640
_E3_BLOCK = 16384
_NW = 32
_ROWS_PER_W = _BS // _NW

_CHUNKS = (
    (0, 128, 0), (128, 128, 0), (256, 128, 0), (384, 16, 0),
    (400, 104, 1), (504, 94, 1),
)


def _proj_body(et_ref, w_ref, o_ref):
    y = lax.dot_general(
        et_ref[...], w_ref[...],
        dimension_numbers=(((0,), (0,)), ((), ())),
        preferred_element_type=jnp.float32)
    y3 = y.reshape(_E3_BLOCK // 8, 8, 16)
    o_ref[...] = jnp.concatenate([y3[:, k, :] for k in range(8)], axis=1)


def _c12_body(e1_ref, e2_ref, w_ref, o_ref):
    w = w_ref[...]
    p1 = jnp.dot(e1_ref[...], w, preferred_element_type=jnp.float32)
    o_ref[0:104, :] = jnp.concatenate(
        [p1, jnp.zeros((4, 16), jnp.float32)], axis=0)
    o_ref[104:10104, :] = jnp.dot(e2_ref[...], w,
                                  preferred_element_type=jnp.float32)
    o_ref[10104:10112, :] = jnp.zeros((8, 16), jnp.float32)


def _idx_body(m_ref, o_ref):
    m = m_ref[...]
    rows = m.shape[0]
    bi = _C12_BI_OFF + m[:, :199] + 100 * m[:, 1:200]
    tri = m[:, :198] + 100 * m[:, 1:199] + 10000 * m[:, 2:200]
    pad1 = jnp.full((rows, 1), _C12_ZERO, jnp.int32)
    pad2 = jnp.full((rows, _IDX_W - 598), _C12_ZERO, jnp.int32)
    o_ref[...] = jnp.concatenate([m, bi, pad1, tri, pad2], axis=1)


def _sc_body(c12_hbm, c3_hbm, idx_hbm, b_hbm, out_hbm,
             idx_v, rows_v, outblk_v, b_v, sem):
    pltpu.sync_copy(b_hbm, b_v)
    rows_v[598, :] = jnp.zeros((16,), jnp.float32)
    rows_v[599, :] = jnp.zeros((16,), jnp.float32)
    wid = lax.axis_index("s") * 2 + lax.axis_index("c")
    base = wid * _ROWS_PER_W

    @pl.loop(0, _ROWS_PER_W)
    def _(i):
        pltpu.sync_copy(idx_hbm.at[base + i], idx_v)
        tabs = (c12_hbm, c3_hbm)
        copies = [
            pltpu.async_copy(tabs[t].at[idx_v.at[pl.ds(off, n)]],
                             rows_v.at[pl.ds(off, n)], sem)
            for (off, n, t) in _CHUNKS
        ]
        for c in copies:
            c.wait()

        def add4(j, accs):
            a0, a1, a2, a3 = accs
            return (a0 + rows_v[4 * j, :], a1 + rows_v[4 * j + 1, :],
                    a2 + rows_v[4 * j + 2, :], a3 + rows_v[4 * j + 3, :])

        zero = jnp.zeros((16,), jnp.float32)
        a0, a1, a2, a3 = lax.fori_loop(0, 150, add4, (zero, zero, zero, zero))
        acc = (a0 + a1) + (a2 + a3)
        outblk_v[i, :] = acc * jnp.float32(1.0 / _NGRAMS) + b_v[...]

    pltpu.sync_copy(outblk_v, out_hbm.at[pl.ds(base, _ROWS_PER_W)])


def kernel(msg, emb1, emb2, emb3, W, b):
    w16 = jnp.pad(W, ((0, 0), (0, 16 - _ROLES)))
    b16 = jnp.pad(b, (0, 16 - _ROLES))
    e3t = emb3.T

    n3 = emb3.shape[0]
    c3p = pl.pallas_call(
        _proj_body,
        grid=((n3 + _E3_BLOCK - 1) // _E3_BLOCK,),
        in_specs=[
            pl.BlockSpec((_DIM, _E3_BLOCK), lambda i: (0, i)),
            pl.BlockSpec((_DIM, 16), lambda i: (0, 0)),
        ],
        out_specs=pl.BlockSpec((_E3_BLOCK // 8, 128), lambda i: (i, 0)),
        out_shape=jax.ShapeDtypeStruct((n3 // 8, 128), jnp.float32),
    )(e3t, w16)
    c3 = c3p.reshape(n3, 16)

    c12 = pl.pallas_call(
        _c12_body,
        out_shape=jax.ShapeDtypeStruct((_C12_ROWS, 16), jnp.float32),
    )(emb1, emb2, w16)

    idx = pl.pallas_call(
        _idx_body,
        grid=(8,),
        in_specs=[pl.BlockSpec((_BS // 8, _MAX_LEN), lambda i: (i, 0))],
        out_specs=pl.BlockSpec((_BS // 8, _IDX_W), lambda i: (i, 0)),
        out_shape=jax.ShapeDtypeStruct((_BS, _IDX_W), jnp.int32),
    )(msg)

    mesh = plsc.VectorSubcoreMesh(core_axis_name="c", subcore_axis_name="s")
    sc = pl.kernel(
        _sc_body,
        mesh=mesh,
        compiler_params=pltpu.CompilerParams(use_tc_tiling_on_sc=False),
        out_type=jax.ShapeDtypeStruct((_BS, 16), jnp.float32),
        scratch_types=[
            pltpu.VMEM((_IDX_W,), jnp.int32),
            pltpu.VMEM((600, 16), jnp.float32),
            pltpu.VMEM((_ROWS_PER_W, 16), jnp.float32),
            pltpu.VMEM((16,), jnp.float32),
            pltpu.SemaphoreType.DMA,
        ],
    )
    out16 = sc(c12, c3, idx, b16)
    return out16[:, :_ROLES]


# speedup vs baseline: 10.2440x; 1.0019x over previous
import functools

import jax
import jax.numpy as jnp
from jax import lax
from jax.experimental import pallas as pl
from jax.experimental.pallas import tpu as pltpu
from jax.experimental.pallas import tpu_sc as plsc

_VOCAB = 100
_DIM = 64
_ROLES = 8
_BS = 4096
_MAX_LEN = 200

_NGRAMS = 3 * _MAX_LEN - 3
_C12_BI_OFF = 104
_C12_ZERO = 10104
_C12_ROWS = 10112
_IDX_W = 640
_E3_BLOCK = 16384
_NW = 32
_ROWS_PER_W = _BS // _NW

_CHUNKS_A = ((0, 128), (128, 128), (256, 128), (384, 16))
_CHUNKS_B = ((400, 0, 104), (504, 104, 94))


def _proj_body(et_ref, w_ref, o_ref):
    y = lax.dot_general(
        et_ref[...], w_ref[...],
        dimension_numbers=(((0,), (0,)), ((), ())),
        preferred_element_type=jnp.float32)
    y3 = y.reshape(_E3_BLOCK // 8, 8, 16)
    o_ref[...] = jnp.concatenate([y3[:, k, :] for k in range(8)], axis=1)


def _c12_body(e1_ref, e2_ref, w_ref, o_ref):
    w = w_ref[...]
    p1 = jnp.dot(e1_ref[...], w, preferred_element_type=jnp.float32)
    o_ref[0:104, :] = jnp.concatenate(
        [p1, jnp.zeros((4, 16), jnp.float32)], axis=0)
    o_ref[104:10104, :] = jnp.dot(e2_ref[...], w,
                                  preferred_element_type=jnp.float32)
    o_ref[10104:10112, :] = jnp.zeros((8, 16), jnp.float32)


def _idx_body(m_ref, o_ref):
    m = m_ref[...]
    rows = m.shape[0]
    bi = _C12_BI_OFF + m[:, :199] + 100 * m[:, 1:200]
    tri = m[:, :198] + 100 * m[:, 1:199] + 10000 * m[:, 2:200]
    pad1 = jnp.full((rows, 1), _C12_ZERO, jnp.int32)
    pad2 = jnp.full((rows, _IDX_W - 598), _C12_ZERO, jnp.int32)
    o_ref[...] = jnp.concatenate([m, bi, pad1, tri, pad2], axis=1)


def _sca_body(c12_hbm, idx_hbm, outa_hbm, idx_v, rows_v, outblk_v, sem):
    wid = lax.axis_index("s") * 2 + lax.axis_index("c")
    base = wid * _ROWS_PER_W

    @pl.loop(0, _ROWS_PER_W)
    def _(i):
        pltpu.sync_copy(idx_hbm.at[base + i], idx_v)
        copies = [
            pltpu.async_copy(c12_hbm.at[idx_v.at[pl.ds(off, n)]],
                             rows_v.at[pl.ds(off, n)], sem)
            for (off, n) in _CHUNKS_A
        ]
        for c in copies:
            c.wait()

        def add4(j, accs):
            a0, a1, a2, a3 = accs
            return (a0 + rows_v[4 * j, :], a1 + rows_v[4 * j + 1, :],
                    a2 + rows_v[4 * j + 2, :], a3 + rows_v[4 * j + 3, :])

        zero = jnp.zeros((16,), jnp.float32)
        a0, a1, a2, a3 = lax.fori_loop(0, 100, add4, (zero, zero, zero, zero))
        outblk_v[i, :] = (a0 + a1) + (a2 + a3)

    pltpu.sync_copy(outblk_v, outa_hbm.at[pl.ds(base, _ROWS_PER_W)])


def _scb_body(c3_hbm, idx_hbm, sa_hbm, b_hbm, out_hbm,
              idx_v, rows_v, outblk_v, b_v, sem):
    pltpu.sync_copy(b_hbm, b_v)
    rows_v[198, :] = jnp.zeros((16,), jnp.float32)
    rows_v[199, :] = jnp.zeros((16,), jnp.float32)
    wid = lax.axis_index("s") * 2 + lax.axis_index("c")
    base = wid * _ROWS_PER_W
    pltpu.sync_copy(sa_hbm.at[pl.ds(base, _ROWS_PER_W)], outblk_v)

    @pl.loop(0, _ROWS_PER_W)
    def _(i):
        pltpu.sync_copy(idx_hbm.at[base + i], idx_v)
        copies = [
            pltpu.async_copy(c3_hbm.at[idx_v.at[pl.ds(off, n)]],
                             rows_v.at[pl.ds(doff, n)], sem)
            for (off, doff, n) in _CHUNKS_B
        ]
        for c in copies:
            c.wait()

        def add4(j, accs):
            a0, a1, a2, a3 = accs
            return (a0 + rows_v[4 * j, :], a1 + rows_v[4 * j + 1, :],
                    a2 + rows_v[4 * j + 2, :], a3 + rows_v[4 * j + 3, :])

        zero = jnp.zeros((16,), jnp.float32)
        a0, a1, a2, a3 = lax.fori_loop(0, 50, add4, (zero, zero, zero, zero))
        acc = ((a0 + a1) + (a2 + a3)) + outblk_v[i, :]
        outblk_v[i, :] = acc * jnp.float32(1.0 / _NGRAMS) + b_v[...]

    pltpu.sync_copy(outblk_v, out_hbm.at[pl.ds(base, _ROWS_PER_W)])


def kernel(msg, emb1, emb2, emb3, W, b):
    w16 = jnp.pad(W, ((0, 0), (0, 16 - _ROLES)))
    b16 = jnp.pad(b, (0, 16 - _ROLES))
    e3t = emb3.T

    n3 = emb3.shape[0]
    c3p = pl.pallas_call(
        _proj_body,
        grid=((n3 + _E3_BLOCK - 1) // _E3_BLOCK,),
        in_specs=[
            pl.BlockSpec((_DIM, _E3_BLOCK), lambda i: (0, i)),
            pl.BlockSpec((_DIM, 16), lambda i: (0, 0)),
        ],
        out_specs=pl.BlockSpec((_E3_BLOCK // 8, 128), lambda i: (i, 0)),
        out_shape=jax.ShapeDtypeStruct((n3 // 8, 128), jnp.float32),
        compiler_params=pltpu.CompilerParams(
            fuse_transposed_lhs_in_matmul=True),
    )(e3t, w16)
    c3 = c3p.reshape(n3, 16)

    c12 = pl.pallas_call(
        _c12_body,
        out_shape=jax.ShapeDtypeStruct((_C12_ROWS, 16), jnp.float32),
    )(emb1, emb2, w16)

    idx = pl.pallas_call(
        _idx_body,
        grid=(8,),
        in_specs=[pl.BlockSpec((_BS // 8, _MAX_LEN), lambda i: (i, 0))],
        out_specs=pl.BlockSpec((_BS // 8, _IDX_W), lambda i: (i, 0)),
        out_shape=jax.ShapeDtypeStruct((_BS, _IDX_W), jnp.int32),
    )(msg)

    mesh = plsc.VectorSubcoreMesh(core_axis_name="c", subcore_axis_name="s")
    sca = pl.kernel(
        _sca_body,
        mesh=mesh,
        compiler_params=pltpu.CompilerParams(use_tc_tiling_on_sc=False),
        out_type=jax.ShapeDtypeStruct((_BS, 16), jnp.float32),
        scratch_types=[
            pltpu.VMEM((_IDX_W,), jnp.int32),
            pltpu.VMEM((400, 16), jnp.float32),
            pltpu.VMEM((_ROWS_PER_W, 16), jnp.float32),
            pltpu.SemaphoreType.DMA,
        ],
    )
    sa = sca(c12, idx)

    scb = pl.kernel(
        _scb_body,
        mesh=mesh,
        compiler_params=pltpu.CompilerParams(use_tc_tiling_on_sc=False),
        out_type=jax.ShapeDtypeStruct((_BS, 16), jnp.float32),
        scratch_types=[
            pltpu.VMEM((_IDX_W,), jnp.int32),
            pltpu.VMEM((200, 16), jnp.float32),
            pltpu.VMEM((_ROWS_PER_W, 16), jnp.float32),
            pltpu.VMEM((16,), jnp.float32),
            pltpu.SemaphoreType.DMA,
        ],
    )
    out16 = scb(c3, idx, sa, b16)
    return out16[:, :_ROLES]


# speedup vs baseline: 11.9614x; 1.1676x over previous
import functools

import jax
import jax.numpy as jnp
from jax import lax
from jax.experimental import pallas as pl
from jax.experimental.pallas import tpu as pltpu
from jax.experimental.pallas import tpu_sc as plsc

_VOCAB = 100
_DIM = 64
_ROLES = 8
_BS = 4096
_MAX_LEN = 200

_NGRAMS = 3 * _MAX_LEN - 3
_C12_BI_OFF = 104
_C12_ZERO = 10104
_C12_ROWS = 10112
_IDX_W = 640
_E3_BLOCK = 16384
_NW = 32
_ROWS_PER_W = _BS // _NW

_CHUNKS_A = ((0, 128), (128, 128), (256, 128), (384, 16))
_CHUNKS_B = ((400, 0, 104), (504, 104, 94))


def _proj_body(et_ref, w_ref, o_ref):
    y = lax.dot_general(
        et_ref[...], w_ref[...],
        dimension_numbers=(((0,), (0,)), ((), ())),
        preferred_element_type=jnp.float32)
    y3 = y.reshape(_E3_BLOCK // 8, 8, 16)
    o_ref[...] = jnp.concatenate([y3[:, k, :] for k in range(8)], axis=1)


def _c12_body(e1_ref, e2_ref, w_ref, o_ref):
    w = w_ref[...]
    p1 = jnp.dot(e1_ref[...], w, preferred_element_type=jnp.float32)
    o_ref[0:104, :] = jnp.concatenate(
        [p1, jnp.zeros((4, 16), jnp.float32)], axis=0)
    o_ref[104:10104, :] = jnp.dot(e2_ref[...], w,
                                  preferred_element_type=jnp.float32)
    o_ref[10104:10112, :] = jnp.zeros((8, 16), jnp.float32)


def _idx_body(m_ref, o_ref):
    m = m_ref[...]
    rows = m.shape[0]
    bi = _C12_BI_OFF + m[:, :199] + 100 * m[:, 1:200]
    tri = m[:, :198] + 100 * m[:, 1:199] + 10000 * m[:, 2:200]
    pad1 = jnp.full((rows, 1), _C12_ZERO, jnp.int32)
    pad2 = jnp.full((rows, _IDX_W - 598), _C12_ZERO, jnp.int32)
    o_ref[...] = jnp.concatenate([m, bi, pad1, tri, pad2], axis=1)


def _sca_body(c12_hbm, idx_hbm, outa_hbm, idx_v0, idx_v1, rows_v0, rows_v1,
              outblk_v, semi, semg0, semg1):
    idx_b = (idx_v0, idx_v1)
    rows_b = (rows_v0, rows_v1)
    wid = lax.axis_index("s") * 2 + lax.axis_index("c")
    base = wid * _ROWS_PER_W

    def gr(r):
        return jnp.minimum(base + r, _BS - 1)

    def fire_g(p, sem, r):
        for off, n in _CHUNKS_A:
            pltpu.async_copy(c12_hbm.at[idx_b[p].at[pl.ds(off, n)]],
                             rows_b[p].at[pl.ds(off, n)], sem)

    def wait_g(p, sem):
        for off, n in _CHUNKS_A:
            pltpu.make_async_copy(c12_hbm.at[idx_b[p].at[pl.ds(off, n)]],
                                  rows_b[p].at[pl.ds(off, n)], sem).wait()

    def wait_i(p, r):
        pltpu.make_async_copy(idx_hbm.at[gr(r)], idx_b[p], semi).wait()

    def reduce_to(p, i):
        rv = rows_b[p]

        def add4(j, accs):
            a0, a1, a2, a3 = accs
            return (a0 + rv[4 * j, :], a1 + rv[4 * j + 1, :],
                    a2 + rv[4 * j + 2, :], a3 + rv[4 * j + 3, :])
        zero = jnp.zeros((16,), jnp.float32)
        a0, a1, a2, a3 = lax.fori_loop(0, 100, add4, (zero, zero, zero, zero))
        outblk_v[i, :] = (a0 + a1) + (a2 + a3)

    pltpu.sync_copy(idx_hbm.at[base], idx_v0)
    fire_g(0, semg0, 0)
    pltpu.async_copy(idx_hbm.at[gr(1)], idx_v1, semi)

    @pl.loop(0, _ROWS_PER_W // 2)
    def _(i):
        r = 2 * i
        wait_i(1, r + 1)
        fire_g(1, semg1, r + 1)
        wait_g(0, semg0)
        pltpu.async_copy(idx_hbm.at[gr(r + 2)], idx_v0, semi)
        reduce_to(0, r)
        wait_i(0, r + 2)
        fire_g(0, semg0, r + 2)
        wait_g(1, semg1)
        pltpu.async_copy(idx_hbm.at[gr(r + 3)], idx_v1, semi)
        reduce_to(1, r + 1)

    wait_g(0, semg0)
    wait_i(1, 0)
    pltpu.sync_copy(outblk_v, outa_hbm.at[pl.ds(base, _ROWS_PER_W)])


def _scb_body(c3_hbm, idx_hbm, sa_hbm, b_hbm, out_hbm,
              idx_v0, idx_v1, rows_v0, rows_v1, outblk_v, b_v,
              semi, semg0, semg1):
    idx_b = (idx_v0, idx_v1)
    rows_b = (rows_v0, rows_v1)
    pltpu.sync_copy(b_hbm, b_v)
    for rv in rows_b:
        rv[198, :] = jnp.zeros((16,), jnp.float32)
        rv[199, :] = jnp.zeros((16,), jnp.float32)
    wid = lax.axis_index("s") * 2 + lax.axis_index("c")
    base = wid * _ROWS_PER_W
    pltpu.sync_copy(sa_hbm.at[pl.ds(base, _ROWS_PER_W)], outblk_v)

    def gr(r):
        return jnp.minimum(base + r, _BS - 1)

    def fire_g(p, sem):
        for off, doff, n in _CHUNKS_B:
            pltpu.async_copy(c3_hbm.at[idx_b[p].at[pl.ds(off, n)]],
                             rows_b[p].at[pl.ds(doff, n)], sem)

    def wait_g(p, sem):
        for off, doff, n in _CHUNKS_B:
            pltpu.make_async_copy(c3_hbm.at[idx_b[p].at[pl.ds(off, n)]],
                                  rows_b[p].at[pl.ds(doff, n)], sem).wait()

    def wait_i(p, r):
        pltpu.make_async_copy(idx_hbm.at[gr(r)], idx_b[p], semi).wait()

    def reduce_to(p, i):
        rv = rows_b[p]

        def add4(j, accs):
            a0, a1, a2, a3 = accs
            return (a0 + rv[4 * j, :], a1 + rv[4 * j + 1, :],
                    a2 + rv[4 * j + 2, :], a3 + rv[4 * j + 3, :])
        zero = jnp.zeros((16,), jnp.float32)
        a0, a1, a2, a3 = lax.fori_loop(0, 50, add4, (zero, zero, zero, zero))
        acc = ((a0 + a1) + (a2 + a3)) + outblk_v[i, :]
        outblk_v[i, :] = acc * jnp.float32(1.0 / _NGRAMS) + b_v[...]

    pltpu.sync_copy(idx_hbm.at[base], idx_v0)
    fire_g(0, semg0)
    pltpu.async_copy(idx_hbm.at[gr(1)], idx_v1, semi)

    @pl.loop(0, _ROWS_PER_W // 2)
    def _(i):
        r = 2 * i
        wait_i(1, r + 1)
        fire_g(1, semg1)
        wait_g(0, semg0)
        pltpu.async_copy(idx_hbm.at[gr(r + 2)], idx_v0, semi)
        reduce_to(0, r)
        wait_i(0, r + 2)
        fire_g(0, semg0)
        wait_g(1, semg1)
        pltpu.async_copy(idx_hbm.at[gr(r + 3)], idx_v1, semi)
        reduce_to(1, r + 1)

    wait_g(0, semg0)
    wait_i(1, 0)
    pltpu.sync_copy(outblk_v, out_hbm.at[pl.ds(base, _ROWS_PER_W)])


def kernel(msg, emb1, emb2, emb3, W, b):
    w16 = jnp.pad(W, ((0, 0), (0, 16 - _ROLES)))
    b16 = jnp.pad(b, (0, 16 - _ROLES))
    e3t = emb3.T

    n3 = emb3.shape[0]
    c3p = pl.pallas_call(
        _proj_body,
        grid=((n3 + _E3_BLOCK - 1) // _E3_BLOCK,),
        in_specs=[
            pl.BlockSpec((_DIM, _E3_BLOCK), lambda i: (0, i)),
            pl.BlockSpec((_DIM, 16), lambda i: (0, 0)),
        ],
        out_specs=pl.BlockSpec((_E3_BLOCK // 8, 128), lambda i: (i, 0)),
        out_shape=jax.ShapeDtypeStruct((n3 // 8, 128), jnp.float32),
        compiler_params=pltpu.CompilerParams(
            fuse_transposed_lhs_in_matmul=True),
    )(e3t, w16)
    c3 = c3p.reshape(n3, 16)

    c12 = pl.pallas_call(
        _c12_body,
        out_shape=jax.ShapeDtypeStruct((_C12_ROWS, 16), jnp.float32),
    )(emb1, emb2, w16)

    idx = pl.pallas_call(
        _idx_body,
        grid=(8,),
        in_specs=[pl.BlockSpec((_BS // 8, _MAX_LEN), lambda i: (i, 0))],
        out_specs=pl.BlockSpec((_BS // 8, _IDX_W), lambda i: (i, 0)),
        out_shape=jax.ShapeDtypeStruct((_BS, _IDX_W), jnp.int32),
    )(msg)

    mesh = plsc.VectorSubcoreMesh(core_axis_name="c", subcore_axis_name="s")
    sca = pl.kernel(
        _sca_body,
        mesh=mesh,
        compiler_params=pltpu.CompilerParams(use_tc_tiling_on_sc=False),
        out_type=jax.ShapeDtypeStruct((_BS, 16), jnp.float32),
        scratch_types=[
            pltpu.VMEM((_IDX_W,), jnp.int32),
            pltpu.VMEM((_IDX_W,), jnp.int32),
            pltpu.VMEM((400, 16), jnp.float32),
            pltpu.VMEM((400, 16), jnp.float32),
            pltpu.VMEM((_ROWS_PER_W, 16), jnp.float32),
            pltpu.SemaphoreType.DMA,
            pltpu.SemaphoreType.DMA,
            pltpu.SemaphoreType.DMA,
        ],
    )
    sa = sca(c12, idx)

    scb = pl.kernel(
        _scb_body,
        mesh=mesh,
        compiler_params=pltpu.CompilerParams(use_tc_tiling_on_sc=False),
        out_type=jax.ShapeDtypeStruct((_BS, 16), jnp.float32),
        scratch_types=[
            pltpu.VMEM((_IDX_W,), jnp.int32),
            pltpu.VMEM((_IDX_W,), jnp.int32),
            pltpu.VMEM((200, 16), jnp.float32),
            pltpu.VMEM((200, 16), jnp.float32),
            pltpu.VMEM((_ROWS_PER_W, 16), jnp.float32),
            pltpu.VMEM((16,), jnp.float32),
            pltpu.SemaphoreType.DMA,
            pltpu.SemaphoreType.DMA,
            pltpu.SemaphoreType.DMA,
        ],
    )
    out16 = scb(c3, idx, sa, b16)
    return out16[:, :_ROLES]


# speedup vs baseline: 16.3529x; 1.3671x over previous
import functools

import jax
import jax.numpy as jnp
from jax import lax
from jax.experimental import pallas as pl
from jax.experimental.pallas import tpu as pltpu
from jax.experimental.pallas import tpu_sc as plsc

_VOCAB = 100
_DIM = 64
_ROLES = 8
_BS = 4096
_MAX_LEN = 200

_NGRAMS = 3 * _MAX_LEN - 3
_C12_BI_OFF = 104
_C12_ZERO = 10104
_C12_ROWS = 10112
_C12_REPS = 8
_IDX_W = 640
_E3_BLOCK = 16384
_NW = 32
_ROWS_PER_W = _BS // _NW

_CHUNKS_A = ((0, 128), (128, 128), (256, 128), (384, 16))
_CHUNKS_B = ((400, 0, 104), (504, 104, 94))


def _proj_body(et_ref, w_ref, o_ref):
    y = lax.dot_general(
        et_ref[...], w_ref[...],
        dimension_numbers=(((0,), (0,)), ((), ())),
        preferred_element_type=jnp.float32)
    y3 = y.reshape(_E3_BLOCK // 8, 8, 16)
    o_ref[...] = jnp.concatenate([y3[:, k, :] for k in range(8)], axis=1)


def _c12_body(e1_ref, e2_ref, w_ref, o_ref):
    w = w_ref[...]
    p1 = jnp.dot(e1_ref[...], w, preferred_element_type=jnp.float32)
    o_ref[0:104, :] = jnp.concatenate(
        [p1, jnp.zeros((4, 16), jnp.float32)], axis=0)
    o_ref[104:10104, :] = jnp.dot(e2_ref[...], w,
                                  preferred_element_type=jnp.float32)
    o_ref[10104:10112, :] = jnp.zeros((8, 16), jnp.float32)


def _idx_body(m_ref, o_ref):
    m = m_ref[...]
    rows = m.shape[0]
    bi = _C12_BI_OFF + m[:, :199] + 100 * m[:, 1:200]
    tri = m[:, :198] + 100 * m[:, 1:199] + 10000 * m[:, 2:200]
    pad1 = jnp.full((rows, 1), _C12_ZERO, jnp.int32)
    pad2 = jnp.full((rows, _IDX_W - 598), _C12_ZERO, jnp.int32)
    o_ref[...] = jnp.concatenate([m, bi, pad1, tri, pad2], axis=1)


def _sca_body(c12_hbm, idx_hbm, outa_hbm, idx_v0, idx_v1, rows_v0, rows_v1,
              outblk_v, semi, semg0, semg1):
    wid = lax.axis_index("s") * 2 + lax.axis_index("c")
    base = wid * _ROWS_PER_W
    idx_b = (idx_v0, idx_v1)
    rows_b = (rows_v0, rows_v1)
    rep_off = (wid % _C12_REPS) * _C12_ROWS

    def gr(r):
        return jnp.minimum(base + r, _BS - 1)

    def adjust(p):
        iv = idx_b[p]
        roff = jnp.full((16,), rep_off, jnp.int32)

        @pl.loop(0, 25)
        def _(j):
            iv[pl.ds(16 * j, 16)] = iv[pl.ds(16 * j, 16)] + roff

    def fire_g(p, sem, r):
        for off, n in _CHUNKS_A:
            pltpu.async_copy(c12_hbm.at[idx_b[p].at[pl.ds(off, n)]],
                             rows_b[p].at[pl.ds(off, n)], sem)

    def wait_g(p, sem):
        for off, n in _CHUNKS_A:
            pltpu.make_async_copy(c12_hbm.at[idx_b[p].at[pl.ds(off, n)]],
                                  rows_b[p].at[pl.ds(off, n)], sem).wait()

    def wait_i(p, r):
        pltpu.make_async_copy(idx_hbm.at[gr(r)], idx_b[p], semi).wait()

    def reduce_to(p, i):
        rv = rows_b[p]

        def add4(j, accs):
            a0, a1, a2, a3 = accs
            return (a0 + rv[4 * j, :], a1 + rv[4 * j + 1, :],
                    a2 + rv[4 * j + 2, :], a3 + rv[4 * j + 3, :])
        zero = jnp.zeros((16,), jnp.float32)
        a0, a1, a2, a3 = lax.fori_loop(0, 100, add4, (zero, zero, zero, zero))
        outblk_v[i, :] = (a0 + a1) + (a2 + a3)

    pltpu.sync_copy(idx_hbm.at[base], idx_v0)
    adjust(0)
    fire_g(0, semg0, 0)
    pltpu.async_copy(idx_hbm.at[gr(1)], idx_v1, semi)

    @pl.loop(0, _ROWS_PER_W // 2)
    def _(i):
        r = 2 * i
        wait_i(1, r + 1)
        adjust(1)
        fire_g(1, semg1, r + 1)
        wait_g(0, semg0)
        pltpu.async_copy(idx_hbm.at[gr(r + 2)], idx_v0, semi)
        reduce_to(0, r)
        wait_i(0, r + 2)
        adjust(0)
        fire_g(0, semg0, r + 2)
        wait_g(1, semg1)
        pltpu.async_copy(idx_hbm.at[gr(r + 3)], idx_v1, semi)
        reduce_to(1, r + 1)

    wait_g(0, semg0)
    wait_i(1, 0)
    pltpu.sync_copy(outblk_v, outa_hbm.at[pl.ds(base, _ROWS_PER_W)])


def _scb_body(c3_hbm, idx_hbm, sa_hbm, b_hbm, out_hbm,
              idx_v0, idx_v1, rows_v0, rows_v1, outblk_v, b_v,
              semi, semg0, semg1):
    idx_b = (idx_v0, idx_v1)
    rows_b = (rows_v0, rows_v1)
    pltpu.sync_copy(b_hbm, b_v)
    for rv in rows_b:
        rv[198, :] = jnp.zeros((16,), jnp.float32)
        rv[199, :] = jnp.zeros((16,), jnp.float32)
    wid = lax.axis_index("s") * 2 + lax.axis_index("c")
    base = wid * _ROWS_PER_W
    pltpu.sync_copy(sa_hbm.at[pl.ds(base, _ROWS_PER_W)], outblk_v)

    def gr(r):
        return jnp.minimum(base + r, _BS - 1)

    def fire_g(p, sem):
        for off, doff, n in _CHUNKS_B:
            pltpu.async_copy(c3_hbm.at[idx_b[p].at[pl.ds(off, n)]],
                             rows_b[p].at[pl.ds(doff, n)], sem)

    def wait_g(p, sem):
        for off, doff, n in _CHUNKS_B:
            pltpu.make_async_copy(c3_hbm.at[idx_b[p].at[pl.ds(off, n)]],
                                  rows_b[p].at[pl.ds(doff, n)], sem).wait()

    def wait_i(p, r):
        pltpu.make_async_copy(idx_hbm.at[gr(r)], idx_b[p], semi).wait()

    def reduce_to(p, i):
        rv = rows_b[p]

        def add4(j, accs):
            a0, a1, a2, a3 = accs
            return (a0 + rv[4 * j, :], a1 + rv[4 * j + 1, :],
                    a2 + rv[4 * j + 2, :], a3 + rv[4 * j + 3, :])
        zero = jnp.zeros((16,), jnp.float32)
        a0, a1, a2, a3 = lax.fori_loop(0, 50, add4, (zero, zero, zero, zero))
        acc = ((a0 + a1) + (a2 + a3)) + outblk_v[i, :]
        outblk_v[i, :] = acc * jnp.float32(1.0 / _NGRAMS) + b_v[...]

    pltpu.sync_copy(idx_hbm.at[base], idx_v0)
    fire_g(0, semg0)
    pltpu.async_copy(idx_hbm.at[gr(1)], idx_v1, semi)

    @pl.loop(0, _ROWS_PER_W // 2)
    def _(i):
        r = 2 * i
        wait_i(1, r + 1)
        fire_g(1, semg1)
        wait_g(0, semg0)
        pltpu.async_copy(idx_hbm.at[gr(r + 2)], idx_v0, semi)
        reduce_to(0, r)
        wait_i(0, r + 2)
        fire_g(0, semg0)
        wait_g(1, semg1)
        pltpu.async_copy(idx_hbm.at[gr(r + 3)], idx_v1, semi)
        reduce_to(1, r + 1)

    wait_g(0, semg0)
    wait_i(1, 0)
    pltpu.sync_copy(outblk_v, out_hbm.at[pl.ds(base, _ROWS_PER_W)])


def kernel(msg, emb1, emb2, emb3, W, b):
    w16 = jnp.pad(W, ((0, 0), (0, 16 - _ROLES)))
    b16 = jnp.pad(b, (0, 16 - _ROLES))
    e3t = emb3.T

    n3 = emb3.shape[0]
    c3p = pl.pallas_call(
        _proj_body,
        grid=((n3 + _E3_BLOCK - 1) // _E3_BLOCK,),
        in_specs=[
            pl.BlockSpec((_DIM, _E3_BLOCK), lambda i: (0, i)),
            pl.BlockSpec((_DIM, 16), lambda i: (0, 0)),
        ],
        out_specs=pl.BlockSpec((_E3_BLOCK // 8, 128), lambda i: (i, 0)),
        out_shape=jax.ShapeDtypeStruct((n3 // 8, 128), jnp.float32),
        compiler_params=pltpu.CompilerParams(
            fuse_transposed_lhs_in_matmul=True),
    )(e3t, w16)
    c3 = c3p.reshape(n3, 16)

    c12 = pl.pallas_call(
        _c12_body,
        grid=(_C12_REPS,),
        in_specs=[
            pl.BlockSpec((_VOCAB, _DIM), lambda i: (0, 0)),
            pl.BlockSpec((_VOCAB ** 2, _DIM), lambda i: (0, 0)),
            pl.BlockSpec((_DIM, 16), lambda i: (0, 0)),
        ],
        out_specs=pl.BlockSpec((_C12_ROWS, 16), lambda i: (i, 0)),
        out_shape=jax.ShapeDtypeStruct((_C12_REPS * _C12_ROWS, 16),
                                       jnp.float32),
    )(emb1, emb2, w16)

    idx = pl.pallas_call(
        _idx_body,
        grid=(8,),
        in_specs=[pl.BlockSpec((_BS // 8, _MAX_LEN), lambda i: (i, 0))],
        out_specs=pl.BlockSpec((_BS // 8, _IDX_W), lambda i: (i, 0)),
        out_shape=jax.ShapeDtypeStruct((_BS, _IDX_W), jnp.int32),
    )(msg)

    mesh = plsc.VectorSubcoreMesh(core_axis_name="c", subcore_axis_name="s")
    sca = pl.kernel(
        _sca_body,
        mesh=mesh,
        compiler_params=pltpu.CompilerParams(use_tc_tiling_on_sc=False),
        out_type=jax.ShapeDtypeStruct((_BS, 16), jnp.float32),
        scratch_types=[
            pltpu.VMEM((_IDX_W,), jnp.int32),
            pltpu.VMEM((_IDX_W,), jnp.int32),
            pltpu.VMEM((400, 16), jnp.float32),
            pltpu.VMEM((400, 16), jnp.float32),
            pltpu.VMEM((_ROWS_PER_W, 16), jnp.float32),
            pltpu.SemaphoreType.DMA,
            pltpu.SemaphoreType.DMA,
            pltpu.SemaphoreType.DMA,
        ],
    )
    sa = sca(c12, idx)

    scb = pl.kernel(
        _scb_body,
        mesh=mesh,
        compiler_params=pltpu.CompilerParams(use_tc_tiling_on_sc=False),
        out_type=jax.ShapeDtypeStruct((_BS, 16), jnp.float32),
        scratch_types=[
            pltpu.VMEM((_IDX_W,), jnp.int32),
            pltpu.VMEM((_IDX_W,), jnp.int32),
            pltpu.VMEM((200, 16), jnp.float32),
            pltpu.VMEM((200, 16), jnp.float32),
            pltpu.VMEM((_ROWS_PER_W, 16), jnp.float32),
            pltpu.VMEM((16,), jnp.float32),
            pltpu.SemaphoreType.DMA,
            pltpu.SemaphoreType.DMA,
            pltpu.SemaphoreType.DMA,
        ],
    )
    out16 = scb(c3, idx, sa, b16)
    return out16[:, :_ROLES]
